# baseline (device time: 54349 ns/iter reference)
import jax
import jax.numpy as jnp
from jax import lax
from jax.experimental import pallas as pl
from jax.experimental.pallas import tpu as pltpu

N_DEV = 8

G_TBL = (0, 1, 3, 2, 4, 5, 7, 6)
AXIS_OF = {1: 1, 2: 3, 3: 3, 4: 4, 5: 4, 6: 4, 7: 4}
SRC_OF = {1: 0, 2: 0, 3: 1, 4: 0, 5: 1, 6: 2, 7: 3}
TRIGGER = {0: (1, 2, 4), 1: (3, 5), 2: (6,), 3: (7,)}
WAIT_ORDER = (1, 2, 4, 3, 5, 6, 7)


def kernel(A, B):
    m_per, k = A.shape
    _, n = B.shape

    def body(a_ref, b_ref, out_ref, g_ref, sc_ref,
             dsend, drecv, ssend, srecv):
        my = lax.axis_index("i")

        a = a_ref[:, :]
        amax = jnp.max(jnp.abs(a), axis=(0, 1), keepdims=True)
        scale = jnp.maximum(amax, 1e-20) * (1.0 / 127.0)
        q = jnp.clip(jnp.round(a / scale), -127.0, 127.0).astype(jnp.int8)
        g_ref[0, :, :] = q
        sc_ref[0, :, :] = scale

        barrier_sem = pltpu.get_barrier_semaphore()
        for mask in (1, 3, 4):
            pl.semaphore_signal(
                barrier_sem, inc=1,
                device_id=(jnp.bitwise_xor(my, mask),),
                device_id_type=pl.DeviceIdType.MESH,
            )
        pl.semaphore_wait(barrier_sem, 3)

        b_bf = b_ref[:, :].astype(jnp.bfloat16)

        def make(b):
            partner = jnp.bitwise_xor(my, AXIS_OF[b])
            return pltpu.make_async_remote_copy(
                src_ref=g_ref.at[SRC_OF[b]], dst_ref=g_ref.at[b],
                send_sem=dsend.at[b], recv_sem=drecv.at[b],
                device_id=(partner,), device_id_type=pl.DeviceIdType.MESH,
            )

        def make_sc(stage, mask, lo, sz):
            partner = jnp.bitwise_xor(my, mask)
            return pltpu.make_async_remote_copy(
                src_ref=sc_ref.at[pl.ds(0, sz)],
                dst_ref=sc_ref.at[pl.ds(lo, sz)],
                send_sem=ssend.at[stage], recv_sem=srecv.at[stage],
                device_id=(partner,), device_id_type=pl.DeviceIdType.MESH,
            )

        rdmas = {}

        def start_sends(src_slot):
            for b in TRIGGER.get(src_slot, ()):
                rdmas[b] = make(b)
                rdmas[b].start()

        def dot_block(slot):
            aq = g_ref[slot, :, :].astype(jnp.bfloat16) * (
                sc_ref[slot, :, :].astype(jnp.bfloat16)
            )
            c = jnp.dot(aq, b_bf, preferred_element_type=jnp.float32)
            origin = jnp.bitwise_xor(my, G_TBL[slot])
            out_ref[pl.ds(origin * m_per, m_per), :] = c.astype(jnp.bfloat16)

        start_sends(0)
        sc_a = make_sc(0, 1, 1, 1)
        sc_a.start()
        out_ref[pl.ds(my * m_per, m_per), :] = jnp.dot(
            a.astype(jnp.bfloat16), b_bf, preferred_element_type=jnp.float32
        ).astype(jnp.bfloat16)

        rdmas[1].wait_recv()
        sc_a.wait_recv()
        sc_b = make_sc(1, 3, 2, 2)
        sc_b.start()
        start_sends(1)
        dot_block(1)

        rdmas[2].wait_recv()
        sc_b.wait_recv()
        sc_c = make_sc(2, 4, 4, 4)
        sc_c.start()
        start_sends(2)
        dot_block(2)

        rdmas[4].wait_recv()
        sc_c.wait_recv()
        dot_block(4)

        for b in (3, 5, 6, 7):
            rdmas[b].wait_recv()
            start_sends(b)
            dot_block(b)

        for b in WAIT_ORDER:
            rdmas[b].wait_send()
        for s in (sc_a, sc_b, sc_c):
            s.wait_send()

    return pl.pallas_call(
        body,
        out_shape=jax.ShapeDtypeStruct((N_DEV * m_per, n), jnp.bfloat16),
        in_specs=[
            pl.BlockSpec(memory_space=pltpu.VMEM),
            pl.BlockSpec(memory_space=pltpu.VMEM),
        ],
        out_specs=pl.BlockSpec(memory_space=pltpu.VMEM),
        scratch_shapes=[
            pltpu.VMEM((N_DEV, m_per, k), jnp.int8),
            pltpu.VMEM((N_DEV, 1, 1), jnp.float32),
            pltpu.SemaphoreType.DMA((N_DEV,)),
            pltpu.SemaphoreType.DMA((N_DEV,)),
            pltpu.SemaphoreType.DMA((3,)),
            pltpu.SemaphoreType.DMA((3,)),
        ],
        compiler_params=pltpu.CompilerParams(
            collective_id=0, vmem_limit_bytes=100 * 1024 * 1024
        ),
    )(A, B)


# device time: 50688 ns/iter; 1.0722x vs baseline; 1.0722x over previous
import jax
import jax.numpy as jnp
from jax import lax
from jax.experimental import pallas as pl
from jax.experimental.pallas import tpu as pltpu

N_DEV = 8

SRC_OF = {1: 0, 2: 0, 3: 1, 4: 0, 5: 1, 6: 2, 7: 3}
TRIGGER = {0: (1, 2, 4), 1: (3, 5), 2: (6,), 3: (7,)}
WAIT_ORDER = (1, 2, 4, 3, 5, 6, 7)


def _tables(m1, m2, m3):
    slot_xor = (0, m1, m2, m1 ^ m2, m3, m3 ^ m1, m3 ^ m2, m3 ^ m1 ^ m2)
    axis_of = {1: m1, 2: m2, 3: m2, 4: m3, 5: m3, 6: m3, 7: m3}
    return slot_xor, axis_of

XA, AXA = _tables(1, 3, 4)
XB, AXB = _tables(4, 3, 1)
A_SLOT_OF_DIFF = {XA[s]: s for s in range(N_DEV)}


def kernel(A, B):
    m_per, k = A.shape
    _, n = B.shape
    half = m_per // 2

    def body(a_ref, b_ref, out_ref, g_ref, sc_ref,
             dsend, drecv, ssend, srecv):
        my = lax.axis_index("i")

        a = a_ref[:, :]
        amax = jnp.max(jnp.abs(a), axis=(0, 1), keepdims=True)
        scale = jnp.maximum(amax, 1e-20) * (1.0 / 127.0)
        q = jnp.clip(jnp.round(a / scale), -127.0, 127.0).astype(jnp.int8)
        g_ref[0, :, :] = q
        sc_ref[0, :, :] = scale

        barrier_sem = pltpu.get_barrier_semaphore()
        for mask in (1, 3, 4):
            pl.semaphore_signal(
                barrier_sem, inc=1,
                device_id=(jnp.bitwise_xor(my, mask),),
                device_id_type=pl.DeviceIdType.MESH,
            )
        pl.semaphore_wait(barrier_sem, 3)

        b_bf = b_ref[:, :].astype(jnp.bfloat16)

        def make(part, b):
            axis_of = AXA if part == 0 else AXB
            rows = pl.ds(part * half, half)
            partner = jnp.bitwise_xor(my, axis_of[b])
            return pltpu.make_async_remote_copy(
                src_ref=g_ref.at[SRC_OF[b], rows, :],
                dst_ref=g_ref.at[b, rows, :],
                send_sem=dsend.at[part, b], recv_sem=drecv.at[part, b],
                device_id=(partner,), device_id_type=pl.DeviceIdType.MESH,
            )

        def make_sc(stage, mask, lo, sz):
            partner = jnp.bitwise_xor(my, mask)
            return pltpu.make_async_remote_copy(
                src_ref=sc_ref.at[pl.ds(0, sz)],
                dst_ref=sc_ref.at[pl.ds(lo, sz)],
                send_sem=ssend.at[stage], recv_sem=srecv.at[stage],
                device_id=(partner,), device_id_type=pl.DeviceIdType.MESH,
            )

        rdmas = {}

        def start_sends(part, src_slot):
            for b in TRIGGER.get(src_slot, ()):
                rdmas[(part, b)] = make(part, b)
                rdmas[(part, b)].start()

        def dot_block(part, slot):
            slot_xor = XA if part == 0 else XB
            aq = g_ref[slot, pl.ds(part * half, half), :].astype(
                jnp.bfloat16
            ) * sc_ref[A_SLOT_OF_DIFF[slot_xor[slot]], :, :].astype(
                jnp.bfloat16
            )
            c = jnp.dot(aq, b_bf, preferred_element_type=jnp.float32)
            origin = jnp.bitwise_xor(my, slot_xor[slot])
            out_ref[pl.ds(origin * m_per + part * half, half), :] = (
                c.astype(jnp.bfloat16)
            )

        start_sends(0, 0)
        start_sends(1, 0)
        sc_a = make_sc(0, 1, 1, 1)
        sc_a.start()

        out_ref[pl.ds(my * m_per, m_per), :] = jnp.dot(
            a.astype(jnp.bfloat16), b_bf, preferred_element_type=jnp.float32
        ).astype(jnp.bfloat16)

        sc_a.wait_recv()
        sc_b = make_sc(1, 3, 2, 2)
        sc_b.start()
        sc_b.wait_recv()
        sc_c = make_sc(2, 4, 4, 4)
        sc_c.start()
        sc_c.wait_recv()

        for b in WAIT_ORDER:
            for part in (0, 1):
                rdmas[(part, b)].wait_recv()
                start_sends(part, b)
                dot_block(part, b)

        for key, rd in rdmas.items():
            rd.wait_send()
        for s in (sc_a, sc_b, sc_c):
            s.wait_send()

    return pl.pallas_call(
        body,
        out_shape=jax.ShapeDtypeStruct((N_DEV * m_per, n), jnp.bfloat16),
        in_specs=[
            pl.BlockSpec(memory_space=pltpu.VMEM),
            pl.BlockSpec(memory_space=pltpu.VMEM),
        ],
        out_specs=pl.BlockSpec(memory_space=pltpu.VMEM),
        scratch_shapes=[
            pltpu.VMEM((N_DEV, m_per, k), jnp.int8),
            pltpu.VMEM((N_DEV, 1, 1), jnp.float32),
            pltpu.SemaphoreType.DMA((2, N_DEV)),
            pltpu.SemaphoreType.DMA((2, N_DEV)),
            pltpu.SemaphoreType.DMA((3,)),
            pltpu.SemaphoreType.DMA((3,)),
        ],
        compiler_params=pltpu.CompilerParams(
            collective_id=0, vmem_limit_bytes=100 * 1024 * 1024
        ),
    )(A, B)
